# baseline (speedup 1.0000x reference)
import functools

import jax
import jax.numpy as jnp
from jax.experimental import pallas as pl
from jax.experimental.pallas import tpu as pltpu

BLK = 256
KRES = 8


def _body(adj_ref, x_ref, win_ref, bin_ref, wm1a_ref, wm1b_ref, bm1_ref,
          wm2_ref, bm2_ref, wm3_ref, bm3_ref,
          wr1m_ref, br1m_ref, wr2m_ref, br2m_ref,
          wr1v_ref, br1v_ref, wr2v_ref, br2v_ref,
          zm_ref, zlv_ref,
          abf_scr, h_scr, dinv_scr, pool_scr, *, n_nodes):
    t = pl.program_id(0)
    b = pl.program_id(1)
    i = pl.program_id(2)
    nb = pl.num_programs(2)
    rows = pl.ds(i * BLK, BLK)

    @pl.when(jnp.logical_and(t == 0, i == 0))
    def _():
        xb = x_ref[b]
        h0 = jnp.tanh(
            jnp.dot(xb, win_ref[...], preferred_element_type=jnp.float32)
            + bin_ref[...])
        h_scr[0, b] = h0
        pool_scr[b] = jnp.zeros_like(pool_scr[b])

    @pl.when(t == 0)
    def _():
        a = adj_ref[0]
        s = jnp.sum(a, axis=1, keepdims=True)
        dinv_scr[b, rows] = 1.0 / jnp.maximum(s, 1.0)

    @pl.when(jnp.logical_and(t == 0, i < KRES))
    def _():
        abf_scr[b, rows] = adj_ref[0].astype(jnp.bfloat16)

    src = t % 2
    dst = 1 - src

    def step(a16):
        h_all = h_scr[src, b].astype(jnp.bfloat16)
        m = jnp.dot(a16, h_all, preferred_element_type=jnp.float32)
        m = m * dinv_scr[b, rows]
        h_blk = h_scr[src, b, rows]

        u = jnp.dot(h_blk, wm1a_ref[...], preferred_element_type=jnp.float32)
        u = u + jnp.dot(m, wm1b_ref[...], preferred_element_type=jnp.float32)
        u = jax.nn.relu(u + bm1_ref[...])
        u = jax.nn.relu(
            jnp.dot(u, wm2_ref[...], preferred_element_type=jnp.float32)
            + bm2_ref[...])
        u = jnp.dot(u, wm3_ref[...], preferred_element_type=jnp.float32) + bm3_ref[...]
        h_new = h_blk + u

        @pl.when(t < 2)
        def _():
            h_scr[dst, b, rows] = h_new

        @pl.when(t == 2)
        def _():
            pool_scr[b] = pool_scr[b] + jnp.sum(h_new, axis=0)

    @pl.when(i < KRES)
    def _():
        step(abf_scr[b, rows])

    @pl.when(i >= KRES)
    def _():
        step(adj_ref[0].astype(jnp.bfloat16))

    @pl.when(jnp.logical_and(t == 2, jnp.logical_and(b == pl.num_programs(1) - 1,
                                                     i == nb - 1)))
    def _():
        for bb in range(pl.num_programs(1)):
            pool = (pool_scr[bb] * (1.0 / n_nodes)).reshape(1, -1)

            hm = jax.nn.relu(
                jnp.dot(pool, wr1m_ref[...], preferred_element_type=jnp.float32)
                + br1m_ref[...])
            zm = jnp.dot(hm, wr2m_ref[...], preferred_element_type=jnp.float32) + br2m_ref[...]
            zm_ref[bb] = zm.reshape(-1)

            hv = jax.nn.relu(
                jnp.dot(pool, wr1v_ref[...], preferred_element_type=jnp.float32)
                + br1v_ref[...])
            zlv = jnp.dot(hv, wr2v_ref[...], preferred_element_type=jnp.float32) + br2v_ref[...]
            zlv_ref[bb] = zlv.reshape(-1)


def kernel(x, adj, W_in, b_in, Wm1, bm1, Wm2, bm2, Wm3, bm3,
           Wr1m, br1m, Wr2m, br2m, Wr1v, br1v, Wr2v, br2v):
    B, N, D_IN = x.shape
    D_H = W_in.shape[1]
    D_Z = Wr2m.shape[1]
    nb = N // BLK

    Wm1a, Wm1b = Wm1[:D_H], Wm1[D_H:]

    def full(arr):
        return pl.BlockSpec(arr.shape, lambda t, b, i: (0,) * arr.ndim)

    biases = [b_in, bm1, bm2, bm3, br1m, br2m, br1v, br2v]
    b_in, bm1, bm2, bm3, br1m, br2m, br1v, br2v = [
        v.reshape(1, -1) for v in biases]

    def adj_idx(t, b, i):
        return (b, jnp.where(t == 0, i, jnp.maximum(i, KRES)), 0)

    in_specs = [
        pl.BlockSpec((1, BLK, N), adj_idx),
        full(x),
        full(W_in), full(b_in),
        full(Wm1a), full(Wm1b), full(bm1),
        full(Wm2), full(bm2), full(Wm3), full(bm3),
        full(Wr1m), full(br1m), full(Wr2m), full(br2m),
        full(Wr1v), full(br1v), full(Wr2v), full(br2v),
    ]

    out = pl.pallas_call(
        functools.partial(_body, n_nodes=N),
        grid=(3, B, nb),
        in_specs=in_specs,
        out_specs=[
            pl.BlockSpec((B, D_Z), lambda t, b, i: (0, 0)),
            pl.BlockSpec((B, D_Z), lambda t, b, i: (0, 0)),
        ],
        out_shape=[
            jax.ShapeDtypeStruct((B, D_Z), jnp.float32),
            jax.ShapeDtypeStruct((B, D_Z), jnp.float32),
        ],
        scratch_shapes=[
            pltpu.VMEM((B, KRES * BLK, N), jnp.bfloat16),
            pltpu.VMEM((2, B, N, D_H), jnp.float32),
            pltpu.VMEM((B, N, 1), jnp.float32),
            pltpu.VMEM((B, D_H), jnp.float32),
        ],
        compiler_params=pltpu.CompilerParams(
            dimension_semantics=("arbitrary", "arbitrary", "arbitrary")),
    )(adj, x, W_in, b_in, Wm1a, Wm1b, bm1, Wm2, bm2, Wm3, bm3,
      Wr1m, br1m, Wr2m, br2m, Wr1v, br1v, Wr2v, br2v)
    return (out[0], out[1])


# speedup vs baseline: 1.0502x; 1.0502x over previous
import functools

import jax
import jax.numpy as jnp
from jax.experimental import pallas as pl
from jax.experimental.pallas import tpu as pltpu

BLK = 512
RHS = 64


def _proj_body(x_ref, win_ref, bin_ref, h0_ref, hx0_ref):
    for b in range(x_ref.shape[0]):
        h0 = jnp.tanh(
            jnp.dot(x_ref[b], win_ref[...], preferred_element_type=jnp.float32)
            + bin_ref[...])
        h0_ref[b] = h0
        hx0_ref[b] = _extend(h0)


def _extend(h):
    n, d_h = h.shape
    lane = jax.lax.broadcasted_iota(jnp.int32, (n, RHS), 1)
    hp = jnp.pad(h, ((0, 0), (0, RHS - d_h)))
    return jnp.where(lane == d_h, 1.0, jnp.where(lane < d_h, hp, 0.0)
                     ).astype(jnp.bfloat16)


def _mlp_update(h_blk, m, wm1a_ref, wm1b_ref, bm1_ref, wm2_ref, bm2_ref,
                wm3_ref, bm3_ref):
    u = jnp.dot(h_blk, wm1a_ref[...], preferred_element_type=jnp.float32)
    u = u + jnp.dot(m, wm1b_ref[...], preferred_element_type=jnp.float32)
    u = jax.nn.relu(u + bm1_ref[...])
    u = jax.nn.relu(
        jnp.dot(u, wm2_ref[...], preferred_element_type=jnp.float32)
        + bm2_ref[...])
    u = jnp.dot(u, wm3_ref[...], preferred_element_type=jnp.float32) + bm3_ref[...]
    return h_blk + u


def _iter1_body(adj_ref, h_ref, hx_ref,
                wm1a_ref, wm1b_ref, bm1_ref, wm2_ref, bm2_ref, wm3_ref,
                bm3_ref, abf_ref, hn_ref, hxn_ref):
    b = pl.program_id(0)
    i = pl.program_id(1)
    d_h = h_ref.shape[-1]
    a16 = adj_ref[0].astype(jnp.bfloat16)
    abf_ref[0] = a16
    me = jnp.dot(a16, hx_ref[b], preferred_element_type=jnp.float32)
    dinv = 1.0 / jnp.maximum(me[:, d_h:d_h + 1], 1.0)
    m = me[:, :d_h] * dinv
    h_blk = h_ref[b, pl.ds(i * BLK, BLK)]
    h_new = _mlp_update(h_blk, m, wm1a_ref, wm1b_ref, bm1_ref, wm2_ref,
                        bm2_ref, wm3_ref, bm3_ref)
    hn_ref[0] = h_new
    hxn_ref[0] = _extend(h_new)


def _iter_body(abf_ref, h_ref, hx_ref,
               wm1a_ref, wm1b_ref, bm1_ref, wm2_ref, bm2_ref, wm3_ref,
               bm3_ref, hn_ref, hxn_ref):
    b = pl.program_id(0)
    i = pl.program_id(1)
    d_h = h_ref.shape[-1]
    a16 = abf_ref[0]
    me = jnp.dot(a16, hx_ref[b], preferred_element_type=jnp.float32)
    dinv = 1.0 / jnp.maximum(me[:, d_h:d_h + 1], 1.0)
    m = me[:, :d_h] * dinv
    h_blk = h_ref[b, pl.ds(i * BLK, BLK)]
    h_new = _mlp_update(h_blk, m, wm1a_ref, wm1b_ref, bm1_ref, wm2_ref,
                        bm2_ref, wm3_ref, bm3_ref)
    hn_ref[0] = h_new
    hxn_ref[0] = _extend(h_new)


def _readout_body(h_ref, wr1m_ref, br1m_ref, wr2m_ref, br2m_ref,
                  wr1v_ref, br1v_ref, wr2v_ref, br2v_ref, zm_ref, zlv_ref):
    n = h_ref.shape[1]
    for b in range(h_ref.shape[0]):
        pool = (jnp.sum(h_ref[b], axis=0) * (1.0 / n)).reshape(1, -1)
        hm = jax.nn.relu(
            jnp.dot(pool, wr1m_ref[...], preferred_element_type=jnp.float32)
            + br1m_ref[...])
        zm = jnp.dot(hm, wr2m_ref[...], preferred_element_type=jnp.float32) + br2m_ref[...]
        zm_ref[b] = zm.reshape(-1)
        hv = jax.nn.relu(
            jnp.dot(pool, wr1v_ref[...], preferred_element_type=jnp.float32)
            + br1v_ref[...])
        zlv = jnp.dot(hv, wr2v_ref[...], preferred_element_type=jnp.float32) + br2v_ref[...]
        zlv_ref[b] = zlv.reshape(-1)


def kernel(x, adj, W_in, b_in, Wm1, bm1, Wm2, bm2, Wm3, bm3,
           Wr1m, br1m, Wr2m, br2m, Wr1v, br1v, Wr2v, br2v):
    B, N, D_IN = x.shape
    D_H = W_in.shape[1]
    D_Z = Wr2m.shape[1]
    nb = N // BLK

    Wm1a, Wm1b = Wm1[:D_H], Wm1[D_H:]
    biases = [b_in, bm1, bm2, bm3, br1m, br2m, br1v, br2v]
    b_in, bm1, bm2, bm3, br1m, br2m, br1v, br2v = [
        v.reshape(1, -1) for v in biases]

    f32 = jnp.float32
    bf16 = jnp.bfloat16
    params = pltpu.CompilerParams(dimension_semantics=("arbitrary", "arbitrary"))

    h0, hx0 = pl.pallas_call(
        _proj_body,
        out_shape=[jax.ShapeDtypeStruct((B, N, D_H), f32),
                   jax.ShapeDtypeStruct((B, N, RHS), bf16)],
    )(x, W_in, b_in)

    def full(arr):
        return pl.BlockSpec(arr.shape, lambda b, i: (0,) * arr.ndim)

    wspecs = [full(Wm1a), full(Wm1b), full(bm1), full(Wm2), full(bm2),
              full(Wm3), full(bm3)]
    weights = (Wm1a, Wm1b, bm1, Wm2, bm2, Wm3, bm3)
    rowblk = lambda w: pl.BlockSpec((1, BLK, w), lambda b, i: (b, i, 0))

    adj16, h1, hx1 = pl.pallas_call(
        _iter1_body,
        grid=(B, nb),
        in_specs=[rowblk(N), full(h0), full(hx0)] + wspecs,
        out_specs=[rowblk(N), rowblk(D_H), rowblk(RHS)],
        out_shape=[jax.ShapeDtypeStruct((B, N, N), bf16),
                   jax.ShapeDtypeStruct((B, N, D_H), f32),
                   jax.ShapeDtypeStruct((B, N, RHS), bf16)],
        compiler_params=params,
    )(adj, h0, hx0, *weights)

    def gnn_iter(h, hx):
        return pl.pallas_call(
            _iter_body,
            grid=(B, nb),
            in_specs=[rowblk(N), full(h), full(hx)] + wspecs,
            out_specs=[rowblk(D_H), rowblk(RHS)],
            out_shape=[jax.ShapeDtypeStruct((B, N, D_H), f32),
                       jax.ShapeDtypeStruct((B, N, RHS), bf16)],
            compiler_params=params,
        )(adj16, h, hx, *weights)

    h2, hx2 = gnn_iter(h1, hx1)
    h3, _ = gnn_iter(h2, hx2)

    zm, zlv = pl.pallas_call(
        _readout_body,
        out_shape=[jax.ShapeDtypeStruct((B, D_Z), f32),
                   jax.ShapeDtypeStruct((B, D_Z), f32)],
    )(h3, Wr1m, br1m, Wr2m, br2m, Wr1v, br1v, Wr2v, br2v)
    return (zm, zlv)
